# baseline (speedup 1.0000x reference)
import jax
import jax.numpy as jnp
import numpy as np
from jax import lax
from jax.experimental import pallas as pl
from jax.experimental.pallas import tpu as pltpu
from jax.experimental.pallas import tpu_sc as plsc

X_LIMS = (0.0, 50.0)
Y_LIMS = (-180.0, 180.0)
Z_LIMS = (-5.0, 3.0)

SCALE_PARAMS = (
    (200.0, 720.0, 80.0, 200, 80),
    (100.0, 360.0, 40.0, 100, 40),
)

ATAN_COEFS = (
    0.99999934, -0.3332986, 0.19946562, -0.13908611,
    0.09642154, -0.05591178, 0.02186261, -0.00405448,
)

NUM_CORES = 2
NUM_SUBCORES = 16
NW = NUM_CORES * NUM_SUBCORES
LANES = 16
CHUNK = 1000
CHUNKS_PER_W = 15


def _f32(v):
    return jnp.float32(v)


def _sqrt16(s):
    i = plsc.bitcast(s, jnp.int32)
    r = plsc.bitcast(jnp.int32(0x5F3759DF) - lax.shift_right_arithmetic(i, 1),
                     jnp.float32)
    for _ in range(3):
        r = r * (_f32(1.5) - _f32(0.5) * s * r * r)
    return jnp.where(s <= _f32(1e-35), _f32(0.0), s * r)


def _atan2_16(y, x):
    ax = jnp.abs(x)
    ay = jnp.abs(y)
    den = jnp.maximum(ax, ay)
    num = jnp.minimum(ax, ay)
    t = jnp.where(den == _f32(0.0), _f32(0.0), num / den)
    u = t * t
    p = jnp.full((LANES,), ATAN_COEFS[-1], jnp.float32)
    for c in ATAN_COEFS[-2::-1]:
        p = p * u + _f32(c)
    r = p * t
    r = jnp.where(ay > ax, _f32(np.pi / 2) - r, r)
    xneg = plsc.bitcast(x, jnp.int32) < 0
    r = jnp.where(xneg, _f32(np.pi) - r, r)
    sign_y = plsc.bitcast(y, jnp.int32) & jnp.int32(-2147483648)
    return plsc.bitcast(plsc.bitcast(r, jnp.int32) | sign_y, jnp.float32)


def _quant16(data, lo, hi, size):
    idx = (data - _f32(lo)) / _f32(hi - lo) * _f32(size)
    il = idx.astype(jnp.int32)
    return il, idx - il.astype(jnp.float32)


def _body(pc_hbm, filt_hbm, idxa_hbm, xya_hbm, topa_hbm, idx2a_hbm, xza_hbm,
          fra_hbm, idxb_hbm, xyb_hbm, topb_hbm, idx2b_hbm, xzb_hbm, frb_hbm,
          in_v, filt_v, idxa_v, xya_v, topa_v, idx2a_v, xza_v, fra_v,
          idxb_v, xyb_v, topb_v, idx2b_v, xzb_v, frb_v):
    wid = lax.axis_index("s") * NUM_CORES + lax.axis_index("c")
    per_w = CHUNK * CHUNKS_PER_W
    iota = lax.iota(jnp.int32, LANES)

    def vgroup(off):
        b4 = iota * 4 + off * 4
        x = plsc.load_gather(in_v, [b4])
        y = plsc.load_gather(in_v, [b4 + 1])
        z = plsc.load_gather(in_v, [b4 + 2])
        w = plsc.load_gather(in_v, [b4 + 3])

        rho = _sqrt16(x * x + y * y)
        phi = _atan2_16(y, x) / _f32(np.pi) * _f32(180.0)

        plsc.store_scatter(filt_v, [b4], rho)
        plsc.store_scatter(filt_v, [b4 + 1], phi)
        plsc.store_scatter(filt_v, [b4 + 2], z)
        plsc.store_scatter(filt_v, [b4 + 3], w)

        b2 = iota * 2 + off * 2
        b3 = iota * 3 + off * 3
        outs = ((idxa_v, xya_v, topa_v, idx2a_v, xza_v, fra_v),
                (idxb_v, xyb_v, topb_v, idx2b_v, xzb_v, frb_v))
        for (sx, sy, sz, size, size_z), (idx_v, xy_v, top_v, idx2_v, xz_v,
                                         fr_v) in zip(SCALE_PARAMS, outs):
            xi, xr = _quant16(rho, X_LIMS[0], X_LIMS[1], sx)
            yi, yr = _quant16(phi, Y_LIMS[0], Y_LIMS[1], sy)
            zi, zr = _quant16(z, Z_LIMS[0], Z_LIMS[1], sz)
            yi = jnp.clip(yi, 0, size - 1)
            idx_v[pl.ds(off, LANES)] = xi * size + yi
            idx2_v[pl.ds(off, LANES)] = (xi * (size * size_z) + yi * size_z
                                         + zi)
            plsc.store_scatter(xy_v, [b2], xi)
            plsc.store_scatter(xy_v, [b2 + 1], yi)
            plsc.store_scatter(top_v, [b2], xr)
            plsc.store_scatter(top_v, [b2 + 1], yr)
            plsc.store_scatter(xz_v, [b3], xi)
            plsc.store_scatter(xz_v, [b3 + 1], yi)
            plsc.store_scatter(xz_v, [b3 + 2], zi)
            plsc.store_scatter(fr_v, [b2], xr)
            plsc.store_scatter(fr_v, [b2 + 1], zr)

    def chunk_body(ci, carry):
        base = wid * per_w + ci * CHUNK
        pltpu.sync_copy(pc_hbm.at[pl.ds(base * 4, CHUNK * 4)], in_v)

        def inner(i, c):
            vgroup(i * LANES)
            return c

        lax.fori_loop(0, CHUNK // LANES, inner, 0)
        vgroup(CHUNK - LANES)

        pltpu.sync_copy(filt_v, filt_hbm.at[pl.ds(base * 4, CHUNK * 4)])
        for v, h, k in ((idxa_v, idxa_hbm, 1), (xya_v, xya_hbm, 2),
                        (topa_v, topa_hbm, 2), (idx2a_v, idx2a_hbm, 1),
                        (xza_v, xza_hbm, 3), (fra_v, fra_hbm, 2),
                        (idxb_v, idxb_hbm, 1), (xyb_v, xyb_hbm, 2),
                        (topb_v, topb_hbm, 2), (idx2b_v, idx2b_hbm, 1),
                        (xzb_v, xzb_hbm, 3), (frb_v, frb_hbm, 2)):
            pltpu.sync_copy(v, h.at[pl.ds(base * k, CHUNK * k)])
        return carry

    lax.fori_loop(0, CHUNKS_PER_W, chunk_body, 0)


@jax.jit
def _polar_sc(pc_flat):
    n = pc_flat.shape[0] // 4
    f32, i32 = jnp.float32, jnp.int32
    out_type = [
        jax.ShapeDtypeStruct((n * 4,), f32),
        jax.ShapeDtypeStruct((n,), i32),
        jax.ShapeDtypeStruct((n * 2,), i32),
        jax.ShapeDtypeStruct((n * 2,), f32),
        jax.ShapeDtypeStruct((n,), i32),
        jax.ShapeDtypeStruct((n * 3,), i32),
        jax.ShapeDtypeStruct((n * 2,), f32),
        jax.ShapeDtypeStruct((n,), i32),
        jax.ShapeDtypeStruct((n * 2,), i32),
        jax.ShapeDtypeStruct((n * 2,), f32),
        jax.ShapeDtypeStruct((n,), i32),
        jax.ShapeDtypeStruct((n * 3,), i32),
        jax.ShapeDtypeStruct((n * 2,), f32),
    ]
    scratch = [pltpu.VMEM((CHUNK * 4,), f32),
               pltpu.VMEM((CHUNK * 4,), f32)]
    for _ in range(2):
        scratch += [
            pltpu.VMEM((CHUNK,), i32), pltpu.VMEM((CHUNK * 2,), i32),
            pltpu.VMEM((CHUNK * 2,), f32), pltpu.VMEM((CHUNK,), i32),
            pltpu.VMEM((CHUNK * 3,), i32), pltpu.VMEM((CHUNK * 2,), f32),
        ]
    return pl.kernel(
        _body,
        out_type=out_type,
        mesh=plsc.VectorSubcoreMesh(core_axis_name="c", subcore_axis_name="s"),
        scratch_types=scratch,
        compiler_params=pltpu.CompilerParams(needs_layout_passes=False),
    )(pc_flat)


def _tc_body(pc_ref, filt_ref, idxa_ref, xya_ref, topa_ref, idx2a_ref,
             xza_ref, fra_ref, idxb_ref, xyb_ref, topb_ref, idx2b_ref,
             xzb_ref, frb_ref):
    x = pc_ref[pl.ds(0, 4), :]
    y = pc_ref[pl.ds(4, 4), :]
    z = pc_ref[pl.ds(8, 4), :]
    w = pc_ref[pl.ds(12, 4), :]
    rho = jnp.sqrt(x * x + y * y)
    phi = jnp.arctan2(y, x) / _f32(np.pi) * _f32(180.0)
    filt_ref[:, 0, :] = rho
    filt_ref[:, 1, :] = phi
    filt_ref[:, 2, :] = z
    filt_ref[:, 3, :] = w
    outs = ((idxa_ref, xya_ref, topa_ref, idx2a_ref, xza_ref, fra_ref),
            (idxb_ref, xyb_ref, topb_ref, idx2b_ref, xzb_ref, frb_ref))
    for (sx, sy, sz, size, size_z), (idx_r, xy_r, top_r, idx2_r, xz_r,
                                     fr_r) in zip(SCALE_PARAMS, outs):
        xi, xr = _quant16(rho, X_LIMS[0], X_LIMS[1], sx)
        yi, yr = _quant16(phi, Y_LIMS[0], Y_LIMS[1], sy)
        zi, zr = _quant16(z, Z_LIMS[0], Z_LIMS[1], sz)
        yi = jnp.clip(yi, 0, size - 1)
        idx_r[...] = xi * size + yi
        idx2_r[...] = xi * (size * size_z) + yi * size_z + zi
        xy_r[:, 0, :] = xi
        xy_r[:, 1, :] = yi
        top_r[:, 0, :] = xr
        top_r[:, 1, :] = yr
        xz_r[:, 0, :] = xi
        xz_r[:, 1, :] = yi
        xz_r[:, 2, :] = zi
        fr_r[:, 0, :] = xr
        fr_r[:, 1, :] = zr


@jax.jit
def _polar_tc(pc):
    b, n, _ = pc.shape
    p = 16384
    g = pl.cdiv(n, p)
    f32, i32 = jnp.float32, jnp.int32
    pcf = jnp.transpose(pc, (2, 0, 1)).reshape(4 * b, n)
    bs = lambda k: pl.BlockSpec((b, k, p), lambda i: (0, 0, i))
    bs1 = pl.BlockSpec((b, p), lambda i: (0, i))
    out_shape = [
        jax.ShapeDtypeStruct((b, 4, n), f32),
        jax.ShapeDtypeStruct((b, n), i32),
        jax.ShapeDtypeStruct((b, 2, n), i32),
        jax.ShapeDtypeStruct((b, 2, n), f32),
        jax.ShapeDtypeStruct((b, n), i32),
        jax.ShapeDtypeStruct((b, 3, n), i32),
        jax.ShapeDtypeStruct((b, 2, n), f32),
        jax.ShapeDtypeStruct((b, n), i32),
        jax.ShapeDtypeStruct((b, 2, n), i32),
        jax.ShapeDtypeStruct((b, 2, n), f32),
        jax.ShapeDtypeStruct((b, n), i32),
        jax.ShapeDtypeStruct((b, 3, n), i32),
        jax.ShapeDtypeStruct((b, 2, n), f32),
    ]
    out_specs = [bs(4), bs1, bs(2), bs(2), bs1, bs(3), bs(2),
                 bs1, bs(2), bs(2), bs1, bs(3), bs(2)]
    outs = pl.pallas_call(
        _tc_body,
        grid=(g,),
        in_specs=[pl.BlockSpec((4 * b, p), lambda i: (0, i))],
        out_specs=out_specs,
        out_shape=out_shape,
    )(pcf)
    sw = lambda a: jnp.swapaxes(a, 1, 2)
    (filt, idxa, xya, topa, idx2a, xza, fra,
     idxb, xyb, topb, idx2b, xzb, frb) = outs
    return (sw(filt), idxa, sw(xya), sw(topa), idx2a, sw(xza), sw(fra),
            idxb, sw(xyb), sw(topb), idx2b, sw(xzb), sw(frb))


SC_COLS = 1024
SC_TAIL0 = 117 * 1024
SC_TAIL = 256


def _scfilt_body(pcf_hbm, filt_hbm, in_v, out_v):
    wid = lax.axis_index("s") * NUM_CORES + lax.axis_index("c")

    def do_cols(c0, cols):
        pltpu.sync_copy(pcf_hbm.at[:, pl.ds(c0, cols)],
                        in_v.at[:, pl.ds(0, cols)])
        for b in range(4):
            def inner(g, c, b=b):
                off = g * LANES
                x = in_v[b, pl.ds(off, LANES)]
                y = in_v[4 + b, pl.ds(off, LANES)]
                z = in_v[8 + b, pl.ds(off, LANES)]
                w = in_v[12 + b, pl.ds(off, LANES)]
                rho = _sqrt16(x * x + y * y)
                phi = _atan2_16(y, x) / _f32(np.pi) * _f32(180.0)
                out_v[b, 0, pl.ds(off, LANES)] = rho
                out_v[b, 1, pl.ds(off, LANES)] = phi
                out_v[b, 2, pl.ds(off, LANES)] = z
                out_v[b, 3, pl.ds(off, LANES)] = w
                return c

            lax.fori_loop(0, cols // LANES, inner, 0)
            pltpu.sync_copy(out_v.at[b, :, pl.ds(0, cols)],
                            filt_hbm.at[b, :, pl.ds(c0, cols)])

    def chunk_body(ci, carry):
        chunk = wid + 32 * ci

        @pl.when(chunk < 117)
        def _():
            do_cols(chunk * SC_COLS, SC_COLS)

        return carry

    lax.fori_loop(0, 4, chunk_body, 0)

    @pl.when(wid == 31)
    def _():
        do_cols(SC_TAIL0, SC_TAIL)


def _sc_filt(pcf):
    n = pcf.shape[1]
    f32 = jnp.float32
    return pl.kernel(
        _scfilt_body,
        out_type=jax.ShapeDtypeStruct((4, 4, n), f32),
        mesh=plsc.VectorSubcoreMesh(core_axis_name="c", subcore_axis_name="s"),
        scratch_types=[pltpu.VMEM((16, SC_COLS), f32),
                       pltpu.VMEM((4, 4, SC_COLS), f32)],
        compiler_params=pltpu.CompilerParams(needs_layout_passes=False),
    )(pcf)


def _tc_body12(pc_ref, idxa_ref, xya_ref, topa_ref, idx2a_ref,
               xza_ref, fra_ref, idxb_ref, xyb_ref, topb_ref, idx2b_ref,
               xzb_ref, frb_ref):
    x = pc_ref[pl.ds(0, 4), :]
    y = pc_ref[pl.ds(4, 4), :]
    z = pc_ref[pl.ds(8, 4), :]
    rho = jnp.sqrt(x * x + y * y)
    phi = jnp.arctan2(y, x) / _f32(np.pi) * _f32(180.0)
    outs = ((idxa_ref, xya_ref, topa_ref, idx2a_ref, xza_ref, fra_ref),
            (idxb_ref, xyb_ref, topb_ref, idx2b_ref, xzb_ref, frb_ref))
    for (sx, sy, sz, size, size_z), (idx_r, xy_r, top_r, idx2_r, xz_r,
                                     fr_r) in zip(SCALE_PARAMS, outs):
        xi, xr = _quant16(rho, X_LIMS[0], X_LIMS[1], sx)
        yi, yr = _quant16(phi, Y_LIMS[0], Y_LIMS[1], sy)
        zi, zr = _quant16(z, Z_LIMS[0], Z_LIMS[1], sz)
        yi = jnp.clip(yi, 0, size - 1)
        idx_r[...] = xi * size + yi
        idx2_r[...] = xi * (size * size_z) + yi * size_z + zi
        xy_r[:, 0, :] = xi
        xy_r[:, 1, :] = yi
        top_r[:, 0, :] = xr
        top_r[:, 1, :] = yr
        xz_r[:, 0, :] = xi
        xz_r[:, 1, :] = yi
        xz_r[:, 2, :] = zi
        fr_r[:, 0, :] = xr
        fr_r[:, 1, :] = zr


@jax.jit
def _polar_hybrid(pc):
    b, n, _ = pc.shape
    p = 16384
    g = pl.cdiv(n, p)
    f32, i32 = jnp.float32, jnp.int32
    pcf = jnp.transpose(pc, (2, 0, 1)).reshape(4 * b, n)
    npad = SC_TAIL0 + SC_TAIL
    pcf_pad = jnp.pad(pcf, ((0, 0), (0, npad - n)))
    filt = _sc_filt(pcf_pad)[:, :, :n]
    bs = lambda k: pl.BlockSpec((b, k, p), lambda i: (0, 0, i))
    bs1 = pl.BlockSpec((b, p), lambda i: (0, i))
    out_shape = [
        jax.ShapeDtypeStruct((b, n), i32),
        jax.ShapeDtypeStruct((b, 2, n), i32),
        jax.ShapeDtypeStruct((b, 2, n), f32),
        jax.ShapeDtypeStruct((b, n), i32),
        jax.ShapeDtypeStruct((b, 3, n), i32),
        jax.ShapeDtypeStruct((b, 2, n), f32),
        jax.ShapeDtypeStruct((b, n), i32),
        jax.ShapeDtypeStruct((b, 2, n), i32),
        jax.ShapeDtypeStruct((b, 2, n), f32),
        jax.ShapeDtypeStruct((b, n), i32),
        jax.ShapeDtypeStruct((b, 3, n), i32),
        jax.ShapeDtypeStruct((b, 2, n), f32),
    ]
    out_specs = [bs1, bs(2), bs(2), bs1, bs(3), bs(2),
                 bs1, bs(2), bs(2), bs1, bs(3), bs(2)]
    outs = pl.pallas_call(
        _tc_body12,
        grid=(g,),
        in_specs=[pl.BlockSpec((4 * b, p), lambda i: (0, i))],
        out_specs=out_specs,
        out_shape=out_shape,
    )(pcf)
    sw = lambda a: jnp.swapaxes(a, 1, 2)
    (idxa, xya, topa, idx2a, xza, fra,
     idxb, xyb, topb, idx2b, xzb, frb) = outs
    return (sw(filt), idxa, sw(xya), sw(topa), idx2a, sw(xza), sw(fra),
            idxb, sw(xyb), sw(topb), idx2b, sw(xzb), sw(frb))


def _tc2_body(pc_ref, f_ref, i_ref):
    x = pc_ref[pl.ds(0, 4), :]
    y = pc_ref[pl.ds(4, 4), :]
    z = pc_ref[pl.ds(8, 4), :]
    rho = jnp.sqrt(x * x + y * y)
    phi = jnp.arctan2(y, x) / _f32(np.pi) * _f32(180.0)
    f_ref[:, 0, :] = rho
    f_ref[:, 1, :] = phi
    for s, (sx, sy, sz, size, size_z) in enumerate(SCALE_PARAMS):
        xi, xr = _quant16(rho, X_LIMS[0], X_LIMS[1], sx)
        yi, yr = _quant16(phi, Y_LIMS[0], Y_LIMS[1], sy)
        zi, zr = _quant16(z, Z_LIMS[0], Z_LIMS[1], sz)
        yi = jnp.clip(yi, 0, size - 1)
        f_ref[:, 2 + 3 * s, :] = xr
        f_ref[:, 3 + 3 * s, :] = yr
        f_ref[:, 4 + 3 * s, :] = zr
        i_ref[:, 5 * s + 0, :] = xi
        i_ref[:, 5 * s + 1, :] = yi
        i_ref[:, 5 * s + 2, :] = zi
        i_ref[:, 5 * s + 3, :] = xi * size + yi
        i_ref[:, 5 * s + 4, :] = xi * (size * size_z) + yi * size_z + zi


@jax.jit
def _polar_tc2(pc):
    b, n, _ = pc.shape
    p = 4096
    g = pl.cdiv(n, p)
    f32, i32 = jnp.float32, jnp.int32
    pcf = jnp.transpose(pc, (2, 0, 1)).reshape(4 * b, n)
    fpl, ipl = pl.pallas_call(
        _tc2_body,
        grid=(g,),
        in_specs=[pl.BlockSpec((4 * b, p), lambda i: (0, i))],
        out_specs=[pl.BlockSpec((b, 8, p), lambda i: (0, 0, i)),
                   pl.BlockSpec((b, 10, p), lambda i: (0, 0, i))],
        out_shape=[jax.ShapeDtypeStruct((b, 8, n), f32),
                   jax.ShapeDtypeStruct((b, 10, n), i32)],
    )(pcf)
    st = lambda parts: jnp.stack(parts, axis=-1)
    outs = [st([fpl[:, 0], fpl[:, 1], pc[..., 2], pc[..., 3]])]
    for s in range(2):
        xi, yi, zi = ipl[:, 5 * s], ipl[:, 5 * s + 1], ipl[:, 5 * s + 2]
        xr, yr, zr = fpl[:, 2 + 3 * s], fpl[:, 3 + 3 * s], fpl[:, 4 + 3 * s]
        outs += [ipl[:, 5 * s + 3], st([xi, yi]), st([xr, yr]),
                 ipl[:, 5 * s + 4], st([xi, yi, zi]), st([xr, zr])]
    return tuple(outs)


def kernel(pc):
    return _polar_hybrid(pc)


def _kernel_sc_path(pc):
    b, n = pc.shape[0], pc.shape[1]
    outs = _polar_sc(pc.reshape(-1))
    (filt, idxa, xya, topa, idx2a, xza, fra,
     idxb, xyb, topb, idx2b, xzb, frb) = outs
    return (
        filt.reshape(b, n, 4),
        idxa.reshape(b, n), xya.reshape(b, n, 2), topa.reshape(b, n, 2),
        idx2a.reshape(b, n), xza.reshape(b, n, 3), fra.reshape(b, n, 2),
        idxb.reshape(b, n), xyb.reshape(b, n, 2), topb.reshape(b, n, 2),
        idx2b.reshape(b, n), xzb.reshape(b, n, 3), frb.reshape(b, n, 2),
    )


# speedup vs baseline: 1.3537x; 1.3537x over previous
import jax
import jax.numpy as jnp
import numpy as np
from jax import lax
from jax.experimental import pallas as pl
from jax.experimental.pallas import tpu as pltpu
from jax.experimental.pallas import tpu_sc as plsc

X_LIMS = (0.0, 50.0)
Y_LIMS = (-180.0, 180.0)
Z_LIMS = (-5.0, 3.0)

SCALE_PARAMS = (
    (200.0, 720.0, 80.0, 200, 80),
    (100.0, 360.0, 40.0, 100, 40),
)

ATAN_COEFS = (
    0.99999934, -0.3332986, 0.19946562, -0.13908611,
    0.09642154, -0.05591178, 0.02186261, -0.00405448,
)

NUM_CORES = 2
NUM_SUBCORES = 16
NW = NUM_CORES * NUM_SUBCORES
LANES = 16
CHUNK = 1000
CHUNKS_PER_W = 15


def _f32(v):
    return jnp.float32(v)


def _sqrt16(s):
    i = plsc.bitcast(s, jnp.int32)
    r = plsc.bitcast(jnp.int32(0x5F3759DF) - lax.shift_right_arithmetic(i, 1),
                     jnp.float32)
    for _ in range(3):
        r = r * (_f32(1.5) - _f32(0.5) * s * r * r)
    return jnp.where(s <= _f32(1e-35), _f32(0.0), s * r)


def _atan2_16(y, x):
    ax = jnp.abs(x)
    ay = jnp.abs(y)
    den = jnp.maximum(ax, ay)
    num = jnp.minimum(ax, ay)
    t = jnp.where(den == _f32(0.0), _f32(0.0), num / den)
    u = t * t
    p = jnp.full((LANES,), ATAN_COEFS[-1], jnp.float32)
    for c in ATAN_COEFS[-2::-1]:
        p = p * u + _f32(c)
    r = p * t
    r = jnp.where(ay > ax, _f32(np.pi / 2) - r, r)
    xneg = plsc.bitcast(x, jnp.int32) < 0
    r = jnp.where(xneg, _f32(np.pi) - r, r)
    sign_y = plsc.bitcast(y, jnp.int32) & jnp.int32(-2147483648)
    return plsc.bitcast(plsc.bitcast(r, jnp.int32) | sign_y, jnp.float32)


def _quant16(data, lo, hi, size):
    idx = (data - _f32(lo)) / _f32(hi - lo) * _f32(size)
    il = idx.astype(jnp.int32)
    return il, idx - il.astype(jnp.float32)


def _body(pc_hbm, filt_hbm, idxa_hbm, xya_hbm, topa_hbm, idx2a_hbm, xza_hbm,
          fra_hbm, idxb_hbm, xyb_hbm, topb_hbm, idx2b_hbm, xzb_hbm, frb_hbm,
          in_v, filt_v, idxa_v, xya_v, topa_v, idx2a_v, xza_v, fra_v,
          idxb_v, xyb_v, topb_v, idx2b_v, xzb_v, frb_v):
    wid = lax.axis_index("s") * NUM_CORES + lax.axis_index("c")
    per_w = CHUNK * CHUNKS_PER_W
    iota = lax.iota(jnp.int32, LANES)

    def vgroup(off):
        b4 = iota * 4 + off * 4
        x = plsc.load_gather(in_v, [b4])
        y = plsc.load_gather(in_v, [b4 + 1])
        z = plsc.load_gather(in_v, [b4 + 2])
        w = plsc.load_gather(in_v, [b4 + 3])

        rho = _sqrt16(x * x + y * y)
        phi = _atan2_16(y, x) / _f32(np.pi) * _f32(180.0)

        plsc.store_scatter(filt_v, [b4], rho)
        plsc.store_scatter(filt_v, [b4 + 1], phi)
        plsc.store_scatter(filt_v, [b4 + 2], z)
        plsc.store_scatter(filt_v, [b4 + 3], w)

        b2 = iota * 2 + off * 2
        b3 = iota * 3 + off * 3
        outs = ((idxa_v, xya_v, topa_v, idx2a_v, xza_v, fra_v),
                (idxb_v, xyb_v, topb_v, idx2b_v, xzb_v, frb_v))
        for (sx, sy, sz, size, size_z), (idx_v, xy_v, top_v, idx2_v, xz_v,
                                         fr_v) in zip(SCALE_PARAMS, outs):
            xi, xr = _quant16(rho, X_LIMS[0], X_LIMS[1], sx)
            yi, yr = _quant16(phi, Y_LIMS[0], Y_LIMS[1], sy)
            zi, zr = _quant16(z, Z_LIMS[0], Z_LIMS[1], sz)
            yi = jnp.clip(yi, 0, size - 1)
            idx_v[pl.ds(off, LANES)] = xi * size + yi
            idx2_v[pl.ds(off, LANES)] = (xi * (size * size_z) + yi * size_z
                                         + zi)
            plsc.store_scatter(xy_v, [b2], xi)
            plsc.store_scatter(xy_v, [b2 + 1], yi)
            plsc.store_scatter(top_v, [b2], xr)
            plsc.store_scatter(top_v, [b2 + 1], yr)
            plsc.store_scatter(xz_v, [b3], xi)
            plsc.store_scatter(xz_v, [b3 + 1], yi)
            plsc.store_scatter(xz_v, [b3 + 2], zi)
            plsc.store_scatter(fr_v, [b2], xr)
            plsc.store_scatter(fr_v, [b2 + 1], zr)

    def chunk_body(ci, carry):
        base = wid * per_w + ci * CHUNK
        pltpu.sync_copy(pc_hbm.at[pl.ds(base * 4, CHUNK * 4)], in_v)

        def inner(i, c):
            vgroup(i * LANES)
            return c

        lax.fori_loop(0, CHUNK // LANES, inner, 0)
        vgroup(CHUNK - LANES)

        pltpu.sync_copy(filt_v, filt_hbm.at[pl.ds(base * 4, CHUNK * 4)])
        for v, h, k in ((idxa_v, idxa_hbm, 1), (xya_v, xya_hbm, 2),
                        (topa_v, topa_hbm, 2), (idx2a_v, idx2a_hbm, 1),
                        (xza_v, xza_hbm, 3), (fra_v, fra_hbm, 2),
                        (idxb_v, idxb_hbm, 1), (xyb_v, xyb_hbm, 2),
                        (topb_v, topb_hbm, 2), (idx2b_v, idx2b_hbm, 1),
                        (xzb_v, xzb_hbm, 3), (frb_v, frb_hbm, 2)):
            pltpu.sync_copy(v, h.at[pl.ds(base * k, CHUNK * k)])
        return carry

    lax.fori_loop(0, CHUNKS_PER_W, chunk_body, 0)


@jax.jit
def _polar_sc(pc_flat):
    n = pc_flat.shape[0] // 4
    f32, i32 = jnp.float32, jnp.int32
    out_type = [
        jax.ShapeDtypeStruct((n * 4,), f32),
        jax.ShapeDtypeStruct((n,), i32),
        jax.ShapeDtypeStruct((n * 2,), i32),
        jax.ShapeDtypeStruct((n * 2,), f32),
        jax.ShapeDtypeStruct((n,), i32),
        jax.ShapeDtypeStruct((n * 3,), i32),
        jax.ShapeDtypeStruct((n * 2,), f32),
        jax.ShapeDtypeStruct((n,), i32),
        jax.ShapeDtypeStruct((n * 2,), i32),
        jax.ShapeDtypeStruct((n * 2,), f32),
        jax.ShapeDtypeStruct((n,), i32),
        jax.ShapeDtypeStruct((n * 3,), i32),
        jax.ShapeDtypeStruct((n * 2,), f32),
    ]
    scratch = [pltpu.VMEM((CHUNK * 4,), f32),
               pltpu.VMEM((CHUNK * 4,), f32)]
    for _ in range(2):
        scratch += [
            pltpu.VMEM((CHUNK,), i32), pltpu.VMEM((CHUNK * 2,), i32),
            pltpu.VMEM((CHUNK * 2,), f32), pltpu.VMEM((CHUNK,), i32),
            pltpu.VMEM((CHUNK * 3,), i32), pltpu.VMEM((CHUNK * 2,), f32),
        ]
    return pl.kernel(
        _body,
        out_type=out_type,
        mesh=plsc.VectorSubcoreMesh(core_axis_name="c", subcore_axis_name="s"),
        scratch_types=scratch,
        compiler_params=pltpu.CompilerParams(needs_layout_passes=False),
    )(pc_flat)


def _tc_body(pc_ref, filt_ref, idxa_ref, xya_ref, topa_ref, idx2a_ref,
             xza_ref, fra_ref, idxb_ref, xyb_ref, topb_ref, idx2b_ref,
             xzb_ref, frb_ref):
    x = pc_ref[pl.ds(0, 4), :]
    y = pc_ref[pl.ds(4, 4), :]
    z = pc_ref[pl.ds(8, 4), :]
    w = pc_ref[pl.ds(12, 4), :]
    rho = jnp.sqrt(x * x + y * y)
    phi = jnp.arctan2(y, x) / _f32(np.pi) * _f32(180.0)
    filt_ref[:, 0, :] = rho
    filt_ref[:, 1, :] = phi
    filt_ref[:, 2, :] = z
    filt_ref[:, 3, :] = w
    outs = ((idxa_ref, xya_ref, topa_ref, idx2a_ref, xza_ref, fra_ref),
            (idxb_ref, xyb_ref, topb_ref, idx2b_ref, xzb_ref, frb_ref))
    for (sx, sy, sz, size, size_z), (idx_r, xy_r, top_r, idx2_r, xz_r,
                                     fr_r) in zip(SCALE_PARAMS, outs):
        xi, xr = _quant16(rho, X_LIMS[0], X_LIMS[1], sx)
        yi, yr = _quant16(phi, Y_LIMS[0], Y_LIMS[1], sy)
        zi, zr = _quant16(z, Z_LIMS[0], Z_LIMS[1], sz)
        yi = jnp.clip(yi, 0, size - 1)
        idx_r[...] = xi * size + yi
        idx2_r[...] = xi * (size * size_z) + yi * size_z + zi
        xy_r[:, 0, :] = xi
        xy_r[:, 1, :] = yi
        top_r[:, 0, :] = xr
        top_r[:, 1, :] = yr
        xz_r[:, 0, :] = xi
        xz_r[:, 1, :] = yi
        xz_r[:, 2, :] = zi
        fr_r[:, 0, :] = xr
        fr_r[:, 1, :] = zr


@jax.jit
def _polar_tc(pc):
    b, n, _ = pc.shape
    p = 16384
    g = pl.cdiv(n, p)
    f32, i32 = jnp.float32, jnp.int32
    pcf = jnp.transpose(pc, (2, 0, 1)).reshape(4 * b, n)
    bs = lambda k: pl.BlockSpec((b, k, p), lambda i: (0, 0, i))
    bs1 = pl.BlockSpec((b, p), lambda i: (0, i))
    out_shape = [
        jax.ShapeDtypeStruct((b, 4, n), f32),
        jax.ShapeDtypeStruct((b, n), i32),
        jax.ShapeDtypeStruct((b, 2, n), i32),
        jax.ShapeDtypeStruct((b, 2, n), f32),
        jax.ShapeDtypeStruct((b, n), i32),
        jax.ShapeDtypeStruct((b, 3, n), i32),
        jax.ShapeDtypeStruct((b, 2, n), f32),
        jax.ShapeDtypeStruct((b, n), i32),
        jax.ShapeDtypeStruct((b, 2, n), i32),
        jax.ShapeDtypeStruct((b, 2, n), f32),
        jax.ShapeDtypeStruct((b, n), i32),
        jax.ShapeDtypeStruct((b, 3, n), i32),
        jax.ShapeDtypeStruct((b, 2, n), f32),
    ]
    out_specs = [bs(4), bs1, bs(2), bs(2), bs1, bs(3), bs(2),
                 bs1, bs(2), bs(2), bs1, bs(3), bs(2)]
    outs = pl.pallas_call(
        _tc_body,
        grid=(g,),
        in_specs=[pl.BlockSpec((4 * b, p), lambda i: (0, i))],
        out_specs=out_specs,
        out_shape=out_shape,
    )(pcf)
    sw = lambda a: jnp.swapaxes(a, 1, 2)
    (filt, idxa, xya, topa, idx2a, xza, fra,
     idxb, xyb, topb, idx2b, xzb, frb) = outs
    return (sw(filt), idxa, sw(xya), sw(topa), idx2a, sw(xza), sw(fra),
            idxb, sw(xyb), sw(topb), idx2b, sw(xzb), sw(frb))


SC_COLS = 1024
SC_TAIL0 = 117 * 1024
SC_TAIL = 256


def _scfilt_body(pcf_hbm, filt_hbm, in_v, out_v):
    wid = lax.axis_index("s") * NUM_CORES + lax.axis_index("c")

    def do_cols(c0, cols):
        pltpu.sync_copy(pcf_hbm.at[:, pl.ds(c0, cols)],
                        in_v.at[:, pl.ds(0, cols)])
        for b in range(4):
            def inner(g, c, b=b):
                off = g * LANES
                x = in_v[b, pl.ds(off, LANES)]
                y = in_v[4 + b, pl.ds(off, LANES)]
                z = in_v[8 + b, pl.ds(off, LANES)]
                w = in_v[12 + b, pl.ds(off, LANES)]
                rho = _sqrt16(x * x + y * y)
                phi = _atan2_16(y, x) / _f32(np.pi) * _f32(180.0)
                out_v[b, 0, pl.ds(off, LANES)] = rho
                out_v[b, 1, pl.ds(off, LANES)] = phi
                out_v[b, 2, pl.ds(off, LANES)] = z
                out_v[b, 3, pl.ds(off, LANES)] = w
                return c

            lax.fori_loop(0, cols // LANES, inner, 0)
            pltpu.sync_copy(out_v.at[b, :, pl.ds(0, cols)],
                            filt_hbm.at[b, :, pl.ds(c0, cols)])

    def chunk_body(ci, carry):
        chunk = wid + 32 * ci

        @pl.when(chunk < 117)
        def _():
            do_cols(chunk * SC_COLS, SC_COLS)

        return carry

    lax.fori_loop(0, 4, chunk_body, 0)

    @pl.when(wid == 31)
    def _():
        do_cols(SC_TAIL0, SC_TAIL)


def _sc_filt(pcf):
    n = pcf.shape[1]
    f32 = jnp.float32
    return pl.kernel(
        _scfilt_body,
        out_type=jax.ShapeDtypeStruct((4, 4, n), f32),
        mesh=plsc.VectorSubcoreMesh(core_axis_name="c", subcore_axis_name="s"),
        scratch_types=[pltpu.VMEM((16, SC_COLS), f32),
                       pltpu.VMEM((4, 4, SC_COLS), f32)],
        compiler_params=pltpu.CompilerParams(needs_layout_passes=False),
    )(pcf)


def _tc_body12(pc_ref, idxa_ref, xya_ref, topa_ref, idx2a_ref,
               xza_ref, fra_ref, idxb_ref, xyb_ref, topb_ref, idx2b_ref,
               xzb_ref, frb_ref):
    x = pc_ref[pl.ds(0, 4), :]
    y = pc_ref[pl.ds(4, 4), :]
    z = pc_ref[pl.ds(8, 4), :]
    rho = jnp.sqrt(x * x + y * y)
    phi = jnp.arctan2(y, x) / _f32(np.pi) * _f32(180.0)
    outs = ((idxa_ref, xya_ref, topa_ref, idx2a_ref, xza_ref, fra_ref),
            (idxb_ref, xyb_ref, topb_ref, idx2b_ref, xzb_ref, frb_ref))
    for (sx, sy, sz, size, size_z), (idx_r, xy_r, top_r, idx2_r, xz_r,
                                     fr_r) in zip(SCALE_PARAMS, outs):
        xi, xr = _quant16(rho, X_LIMS[0], X_LIMS[1], sx)
        yi, yr = _quant16(phi, Y_LIMS[0], Y_LIMS[1], sy)
        zi, zr = _quant16(z, Z_LIMS[0], Z_LIMS[1], sz)
        yi = jnp.clip(yi, 0, size - 1)
        idx_r[...] = xi * size + yi
        idx2_r[...] = xi * (size * size_z) + yi * size_z + zi
        xy_r[:, 0, :] = xi
        xy_r[:, 1, :] = yi
        top_r[:, 0, :] = xr
        top_r[:, 1, :] = yr
        xz_r[:, 0, :] = xi
        xz_r[:, 1, :] = yi
        xz_r[:, 2, :] = zi
        fr_r[:, 0, :] = xr
        fr_r[:, 1, :] = zr


@jax.jit
def _polar_hybrid(pc):
    b, n, _ = pc.shape
    p = 16384
    g = pl.cdiv(n, p)
    f32, i32 = jnp.float32, jnp.int32
    pcf = jnp.transpose(pc, (2, 0, 1)).reshape(4 * b, n)
    npad = SC_TAIL0 + SC_TAIL
    pcf_pad = jnp.pad(pcf, ((0, 0), (0, npad - n)))
    filt = _sc_filt(pcf_pad)[:, :, :n]
    bs = lambda k: pl.BlockSpec((b, k, p), lambda i: (0, 0, i))
    bs1 = pl.BlockSpec((b, p), lambda i: (0, i))
    out_shape = [
        jax.ShapeDtypeStruct((b, n), i32),
        jax.ShapeDtypeStruct((b, 2, n), i32),
        jax.ShapeDtypeStruct((b, 2, n), f32),
        jax.ShapeDtypeStruct((b, n), i32),
        jax.ShapeDtypeStruct((b, 3, n), i32),
        jax.ShapeDtypeStruct((b, 2, n), f32),
        jax.ShapeDtypeStruct((b, n), i32),
        jax.ShapeDtypeStruct((b, 2, n), i32),
        jax.ShapeDtypeStruct((b, 2, n), f32),
        jax.ShapeDtypeStruct((b, n), i32),
        jax.ShapeDtypeStruct((b, 3, n), i32),
        jax.ShapeDtypeStruct((b, 2, n), f32),
    ]
    out_specs = [bs1, bs(2), bs(2), bs1, bs(3), bs(2),
                 bs1, bs(2), bs(2), bs1, bs(3), bs(2)]
    outs = pl.pallas_call(
        _tc_body12,
        grid=(g,),
        in_specs=[pl.BlockSpec((4 * b, p), lambda i: (0, i))],
        out_specs=out_specs,
        out_shape=out_shape,
    )(pcf)
    sw = lambda a: jnp.swapaxes(a, 1, 2)
    (idxa, xya, topa, idx2a, xza, fra,
     idxb, xyb, topb, idx2b, xzb, frb) = outs
    return (sw(filt), idxa, sw(xya), sw(topa), idx2a, sw(xza), sw(fra),
            idxb, sw(xyb), sw(topb), idx2b, sw(xzb), sw(frb))


def _tc2_body(pc_ref, f_ref, i_ref):
    x = pc_ref[pl.ds(0, 4), :]
    y = pc_ref[pl.ds(4, 4), :]
    z = pc_ref[pl.ds(8, 4), :]
    rho = jnp.sqrt(x * x + y * y)
    phi = jnp.arctan2(y, x) / _f32(np.pi) * _f32(180.0)
    f_ref[:, 0, :] = rho
    f_ref[:, 1, :] = phi
    for s, (sx, sy, sz, size, size_z) in enumerate(SCALE_PARAMS):
        xi, xr = _quant16(rho, X_LIMS[0], X_LIMS[1], sx)
        yi, yr = _quant16(phi, Y_LIMS[0], Y_LIMS[1], sy)
        zi, zr = _quant16(z, Z_LIMS[0], Z_LIMS[1], sz)
        yi = jnp.clip(yi, 0, size - 1)
        f_ref[:, 2 + 3 * s, :] = xr
        f_ref[:, 3 + 3 * s, :] = yr
        f_ref[:, 4 + 3 * s, :] = zr
        i_ref[:, 5 * s + 0, :] = xi
        i_ref[:, 5 * s + 1, :] = yi
        i_ref[:, 5 * s + 2, :] = zi
        i_ref[:, 5 * s + 3, :] = xi * size + yi
        i_ref[:, 5 * s + 4, :] = xi * (size * size_z) + yi * size_z + zi


@jax.jit
def _polar_tc2(pc):
    b, n, _ = pc.shape
    p = 4096
    g = pl.cdiv(n, p)
    f32, i32 = jnp.float32, jnp.int32
    pcf = jnp.transpose(pc, (2, 0, 1)).reshape(4 * b, n)
    fpl, ipl = pl.pallas_call(
        _tc2_body,
        grid=(g,),
        in_specs=[pl.BlockSpec((4 * b, p), lambda i: (0, i))],
        out_specs=[pl.BlockSpec((b, 8, p), lambda i: (0, 0, i)),
                   pl.BlockSpec((b, 10, p), lambda i: (0, 0, i))],
        out_shape=[jax.ShapeDtypeStruct((b, 8, n), f32),
                   jax.ShapeDtypeStruct((b, 10, n), i32)],
    )(pcf)
    st = lambda parts: jnp.stack(parts, axis=-1)
    outs = [st([fpl[:, 0], fpl[:, 1], pc[..., 2], pc[..., 3]])]
    for s in range(2):
        xi, yi, zi = ipl[:, 5 * s], ipl[:, 5 * s + 1], ipl[:, 5 * s + 2]
        xr, yr, zr = fpl[:, 2 + 3 * s], fpl[:, 3 + 3 * s], fpl[:, 4 + 3 * s]
        outs += [ipl[:, 5 * s + 3], st([xi, yi]), st([xr, yr]),
                 ipl[:, 5 * s + 4], st([xi, yi, zi]), st([xr, zr])]
    return tuple(outs)


def kernel(pc):
    return _polar_tc(pc)


def _kernel_sc_path(pc):
    b, n = pc.shape[0], pc.shape[1]
    outs = _polar_sc(pc.reshape(-1))
    (filt, idxa, xya, topa, idx2a, xza, fra,
     idxb, xyb, topb, idx2b, xzb, frb) = outs
    return (
        filt.reshape(b, n, 4),
        idxa.reshape(b, n), xya.reshape(b, n, 2), topa.reshape(b, n, 2),
        idx2a.reshape(b, n), xza.reshape(b, n, 3), fra.reshape(b, n, 2),
        idxb.reshape(b, n), xyb.reshape(b, n, 2), topb.reshape(b, n, 2),
        idx2b.reshape(b, n), xzb.reshape(b, n, 3), frb.reshape(b, n, 2),
    )
